# Initial kernel scaffold; baseline (speedup 1.0000x reference)
import functools
import math

import jax
import jax.numpy as jnp
from jax import lax
from jax.experimental import pallas as pl
from jax.experimental.pallas import tpu as pltpu
from jax.experimental.pallas import tpu_sc as plsc

MARGIN = 9.0

NC = 2
NS = 16
NW = NC * NS
LANES = 16

NUM_EDGES = 320000
D = 128
DH = D // 2
EDGES_PER_WORKER = NUM_EDGES // NW
CHUNK = 80
NCHUNKS = EDGES_PER_WORKER // CHUNK
GROUPS = CHUNK // LANES


def _sqrt_nr(x):
  i = plsc.bitcast(x, jnp.int32)
  i = jnp.int32(0x5F3759DF) - (i >> 1)
  y = plsc.bitcast(i, jnp.float32)
  xh = x * 0.5
  y = y * (1.5 - xh * y * y)
  y = y * (1.5 - xh * y * y)
  return x * y


def _body(node_hbm, hidx_hbm, tidx_hbm, rel_hbm, cos_hbm, sin_hbm,
          scale_hbm, bias_hbm, out_hbm,
          hidx_v, tidx_v, rel_v, head_v, tail_v, cos_v, sin_v, sb_v,
          out_v, sem_h, sem_t):
  wid = lax.axis_index("s") * NC + lax.axis_index("c")

  pltpu.sync_copy(cos_hbm, cos_v)
  pltpu.sync_copy(sin_hbm, sin_v)
  pltpu.sync_copy(scale_hbm, sb_v.at[0])
  pltpu.sync_copy(bias_hbm, sb_v.at[1])
  scale = sb_v[0, :]
  biasv = sb_v[1, :]

  iota = lax.iota(jnp.int32, LANES)
  worker_base = wid * EDGES_PER_WORKER

  def chunk_body(c, carry):
    base = pl.multiple_of(worker_base + c * CHUNK, 16)
    pltpu.sync_copy(hidx_hbm.at[pl.ds(base, CHUNK)], hidx_v)
    pltpu.sync_copy(tidx_hbm.at[pl.ds(base, CHUNK)], tidx_v)
    pltpu.sync_copy(rel_hbm.at[pl.ds(base, CHUNK)], rel_v)
    cp_h = pltpu.make_async_copy(node_hbm.at[hidx_v], head_v, sem_h)
    cp_t = pltpu.make_async_copy(node_hbm.at[tidx_v], tail_v, sem_t)
    cp_h.start()
    cp_t.start()
    cp_h.wait()
    cp_t.wait()

    for g in range(GROUPS):
      ev = iota + (g * LANES)
      rv = rel_v[pl.ds(g * LANES, LANES)]

      def jbody(j, acc):
        jv = jnp.full((LANES,), 0, jnp.int32) + j
        jv64 = jv + DH
        re_h = plsc.load_gather(head_v, [ev, jv])
        im_h = plsc.load_gather(head_v, [ev, jv64])
        re_t = plsc.load_gather(tail_v, [ev, jv])
        im_t = plsc.load_gather(tail_v, [ev, jv64])
        cosv = plsc.load_gather(cos_v, [rv, jv])
        sinv = plsc.load_gather(sin_v, [rv, jv])
        re_s = re_h * cosv - im_h * sinv
        im_s = re_h * sinv + im_h * cosv
        rd = re_s - re_t
        im_d = im_s - im_t
        d2 = rd * rd + im_d * im_d
        d2 = jnp.maximum(d2, 1e-30)
        return acc + _sqrt_nr(d2)

      acc = lax.fori_loop(0, DH, jbody, jnp.zeros((LANES,), jnp.float32))
      out_v[pl.ds(g * LANES, LANES)] = acc * scale + biasv

    pltpu.sync_copy(out_v, out_hbm.at[pl.ds(base, CHUNK)])
    return carry

  lax.fori_loop(0, NCHUNKS, chunk_body, jnp.int32(0))


def kernel(node_embeddings, edge_index, relation_type, rel_weight,
           temperature, bias):
  phase = rel_weight * (math.pi / MARGIN)
  cos_t = jnp.cos(phase)
  sin_t = jnp.sin(phase)
  h_idx = edge_index[0]
  t_idx = edge_index[1]
  scale_vec = jnp.full((LANES,), -1.0 / DH, jnp.float32) / temperature
  bias_vec = jnp.full((LANES,), 1.0, jnp.float32) * bias

  mesh = plsc.VectorSubcoreMesh(
      core_axis_name="c", subcore_axis_name="s",
      num_cores=NC, num_subcores=NS)
  run = pl.kernel(
      _body,
      out_type=jax.ShapeDtypeStruct((NUM_EDGES,), jnp.float32),
      mesh=mesh,
      scratch_types=[
          pltpu.VMEM((CHUNK,), jnp.int32),
          pltpu.VMEM((CHUNK,), jnp.int32),
          pltpu.VMEM((CHUNK,), jnp.int32),
          pltpu.VMEM((CHUNK, D), jnp.float32),
          pltpu.VMEM((CHUNK, D), jnp.float32),
          pltpu.VMEM((DH, DH), jnp.float32),
          pltpu.VMEM((DH, DH), jnp.float32),
          pltpu.VMEM((2, LANES), jnp.float32),
          pltpu.VMEM((CHUNK,), jnp.float32),
          pltpu.SemaphoreType.DMA,
          pltpu.SemaphoreType.DMA,
      ],
      name="rotate_head_score_sc",
  )
  return run(node_embeddings, h_idx, t_idx, relation_type,
             cos_t, sin_t, scale_vec, bias_vec)


# speedup vs baseline: 1.2639x; 1.2639x over previous
import functools
import math

import jax
import jax.numpy as jnp
from jax import lax
from jax.experimental import pallas as pl
from jax.experimental.pallas import tpu as pltpu
from jax.experimental.pallas import tpu_sc as plsc

MARGIN = 9.0

NC = 2
NS = 16
NW = NC * NS
LANES = 16

NUM_EDGES = 320000
D = 128
DH = D // 2
EDGES_PER_WORKER = NUM_EDGES // NW
CHUNK = 80
NCHUNKS = EDGES_PER_WORKER // CHUNK
GROUPS = CHUNK // LANES


def _sqrt_nr(x):
  i = plsc.bitcast(x, jnp.int32)
  i = jnp.int32(0x5F3759DF) - (i >> 1)
  y = plsc.bitcast(i, jnp.float32)
  xh = x * 0.5
  y = y * (1.5 - xh * y * y)
  y = y * (1.5 - xh * y * y)
  return x * y


def _body(node_hbm, hidx_hbm, tidx_hbm, rel_hbm, cos_hbm, sin_hbm,
          scale_hbm, bias_hbm, out_hbm,
          hidx_v, tidx_v, rel_v, head_v, tail_v, cos_v, sin_v, sb_v,
          out_v, sem_h, sem_t):
  wid = lax.axis_index("s") * NC + lax.axis_index("c")

  pltpu.sync_copy(cos_hbm, cos_v)
  pltpu.sync_copy(sin_hbm, sin_v)
  pltpu.sync_copy(scale_hbm, sb_v.at[0])
  pltpu.sync_copy(bias_hbm, sb_v.at[1])
  scale = sb_v[0, :]
  biasv = sb_v[1, :]

  iota = lax.iota(jnp.int32, LANES)
  worker_base = wid * EDGES_PER_WORKER

  def chunk_body(c, carry):
    base = pl.multiple_of(worker_base + c * CHUNK, 16)
    pltpu.sync_copy(hidx_hbm.at[pl.ds(base, CHUNK)], hidx_v)
    pltpu.sync_copy(tidx_hbm.at[pl.ds(base, CHUNK)], tidx_v)
    pltpu.sync_copy(rel_hbm.at[pl.ds(base, CHUNK)], rel_v)
    cp_h = pltpu.make_async_copy(node_hbm.at[hidx_v], head_v, sem_h)
    cp_t = pltpu.make_async_copy(node_hbm.at[tidx_v], tail_v, sem_t)
    cp_h.start()
    cp_t.start()
    cp_h.wait()
    cp_t.wait()

    for g in range(GROUPS):
      ev = iota + (g * LANES)
      rv = rel_v[pl.ds(g * LANES, LANES)]

      def jbody(j, acc):
        jv = jnp.full((LANES,), 0, jnp.int32) + j
        jv64 = jv + DH
        re_h = plsc.load_gather(head_v, [ev, jv])
        im_h = plsc.load_gather(head_v, [ev, jv64])
        re_t = plsc.load_gather(tail_v, [ev, jv])
        im_t = plsc.load_gather(tail_v, [ev, jv64])
        cosv = plsc.load_gather(cos_v, [rv, jv])
        sinv = plsc.load_gather(sin_v, [rv, jv])
        re_s = re_h * cosv - im_h * sinv
        im_s = re_h * sinv + im_h * cosv
        rd = re_s - re_t
        im_d = im_s - im_t
        d2 = rd * rd + im_d * im_d
        d2 = jnp.maximum(d2, 1e-30)
        return acc + _sqrt_nr(d2)

      acc = lax.fori_loop(0, DH, jbody, jnp.zeros((LANES,), jnp.float32))
      out_v[pl.ds(g * LANES, LANES)] = acc * scale + biasv

    pltpu.sync_copy(out_v, out_hbm.at[pl.ds(base, CHUNK)])
    return carry

  lax.fori_loop(0, NCHUNKS, chunk_body, jnp.int32(0))


def kernel(node_embeddings, edge_index, relation_type, rel_weight,
           temperature, bias):
  phase = rel_weight * (math.pi / MARGIN)
  cos_t = jnp.cos(phase)
  sin_t = jnp.sin(phase)
  h_idx = edge_index[0]
  t_idx = edge_index[1]
  scale_vec = jnp.full((LANES,), -1.0 / DH, jnp.float32) / temperature
  bias_vec = jnp.full((LANES,), 1.0, jnp.float32) * bias

  mesh = plsc.VectorSubcoreMesh(
      core_axis_name="c", subcore_axis_name="s",
      num_cores=NC, num_subcores=NS)
  run = pl.kernel(
      _body,
      out_type=jax.ShapeDtypeStruct((NUM_EDGES,), jnp.float32),
      mesh=mesh,
      scratch_types=[
          pltpu.VMEM((CHUNK,), jnp.int32),
          pltpu.VMEM((CHUNK,), jnp.int32),
          pltpu.VMEM((CHUNK,), jnp.int32),
          pltpu.VMEM((CHUNK, D), jnp.float32),
          pltpu.VMEM((CHUNK, D), jnp.float32),
          pltpu.VMEM((DH, DH), jnp.float32),
          pltpu.VMEM((DH, DH), jnp.float32),
          pltpu.VMEM((2, LANES), jnp.float32),
          pltpu.VMEM((CHUNK,), jnp.float32),
          pltpu.SemaphoreType.DMA,
          pltpu.SemaphoreType.DMA,
      ],
      compiler_params=pltpu.CompilerParams(needs_layout_passes=False),
      name="rotate_head_score_sc",
  )
  return run(node_embeddings, h_idx, t_idx, relation_type,
             cos_t, sin_t, scale_vec, bias_vec)
